# baseline (speedup 1.0000x reference)
import jax
import jax.numpy as jnp
from jax import lax
from jax.experimental import pallas as pl
from jax.experimental.pallas import tpu as pltpu
from jax.experimental.pallas import tpu_sc as plsc

B, L, F = 4096, 20, 7
O_LIN, O_EMB, O = 16, 8, 24
NC, NS, LANES = 2, 16, 16
NW = NC * NS
CK = 512
NCH = B // CK
NSUB = L * NCH
PER_TILE = NSUB // NW
GB = 4
NGB = CK // (LANES * GB)


def _sc_body(x_hbm, m_hbm, wb_hbm, br_hbm, tab_hbm, out_hbm,
             x_v, m_v, wb_v, br_v, tab_v, o_v):
    wid = lax.axis_index("s") * NC + lax.axis_index("c")
    pltpu.sync_copy(wb_hbm, wb_v)
    pltpu.sync_copy(br_hbm, br_v)
    pltpu.sync_copy(tab_hbm, tab_v)

    def subtask(k, carry):
        t = wid * PER_TILE + k
        l = t // NCH
        c = t % NCH
        pltpu.sync_copy(x_hbm.at[l, :, pl.ds(c * CK, CK)], x_v)
        pltpu.sync_copy(m_hbm.at[l, pl.ds(c * CK, CK)], m_v)

        def inner(gb, carry2):
            base = gb * (LANES * GB)
            xm, mv, mapped = [], [], []
            for g in range(GB):
                off = base + g * LANES
                m = m_v[pl.ds(off, LANES)]
                feats = [x_v[f, pl.ds(off, LANES)] for f in range(6)]
                xm.append([feats[f] * m for f in range(5)])
                z = (feats[5] * m).astype(jnp.int32)
                mapped.append(jnp.where((z >= 57) & (z <= 80), z - 56, 0))
                mv.append(m)
            for o in range(O_LIN):
                bo = br_v[pl.ds(o * LANES, LANES)]
                w = [wb_v[pl.ds((o * 5 + f) * LANES, LANES)] for f in range(5)]
                for g in range(GB):
                    acc = bo
                    for f in range(5):
                        acc = acc + xm[g][f] * w[f]
                    o_v[o, pl.ds(base + g * LANES, LANES)] = jnp.maximum(acc, 0.0)
            for g in range(GB):
                m8 = mapped[g] * O_EMB
                for j in range(O_EMB):
                    e = plsc.load_gather(tab_v, [m8 + j])
                    o_v[O_LIN + j, pl.ds(base + g * LANES, LANES)] = e * mv[g]
            return carry2

        lax.fori_loop(0, NGB, inner, 0)
        pltpu.sync_copy(o_v, out_hbm.at[l, :, pl.ds(c * CK, CK)])
        return carry

    lax.fori_loop(0, PER_TILE, subtask, 0)


def kernel(elements_info, elements_mask, W, b, tm_table):
    x_t = jnp.transpose(elements_info, (1, 2, 0))
    m_t = jnp.transpose(elements_mask, (1, 0))
    wb = jnp.repeat(W.reshape(-1), LANES)
    br = jnp.repeat(b, LANES)
    tab = jnp.pad(tm_table.reshape(-1), (0, 56))
    mesh = plsc.VectorSubcoreMesh(core_axis_name="c", subcore_axis_name="s")
    out = pl.kernel(
        _sc_body,
        out_type=jax.ShapeDtypeStruct((L, O, B), jnp.float32),
        mesh=mesh,
        compiler_params=pltpu.CompilerParams(needs_layout_passes=False),
        scratch_types=[
            pltpu.VMEM((F, CK), jnp.float32),
            pltpu.VMEM((CK,), jnp.float32),
            pltpu.VMEM((80 * LANES,), jnp.float32),
            pltpu.VMEM((O_LIN * LANES,), jnp.float32),
            pltpu.VMEM((256,), jnp.float32),
            pltpu.VMEM((O, CK), jnp.float32),
        ],
    )(x_t, m_t, wb, br, tab)
    return jnp.transpose(out, (2, 0, 1))


# speedup vs baseline: 1.0234x; 1.0234x over previous
import jax
import jax.numpy as jnp
from jax import lax
from jax.experimental import pallas as pl
from jax.experimental.pallas import tpu as pltpu
from jax.experimental.pallas import tpu_sc as plsc

B, L, F = 4096, 20, 7
O_LIN, O_EMB, O = 16, 8, 24
NC, NS, LANES = 2, 16, 16
NW = NC * NS
CK = 512
NCH = B // CK
NSUB = L * NCH
PER_TILE = NSUB // NW
GB = 8
NGB = CK // (LANES * GB)


def _sc_body(x_hbm, m_hbm, wb_hbm, br_hbm, tab_hbm, out_hbm,
             x_v, m_v, wb_v, br_v, tab_v, o_v):
    wid = lax.axis_index("s") * NC + lax.axis_index("c")
    pltpu.sync_copy(wb_hbm, wb_v)
    pltpu.sync_copy(br_hbm, br_v)
    pltpu.sync_copy(tab_hbm, tab_v)

    def subtask(k, carry):
        t = wid * PER_TILE + k
        l = t // NCH
        c = t % NCH
        pltpu.sync_copy(x_hbm.at[l, :, pl.ds(c * CK, CK)], x_v)
        pltpu.sync_copy(m_hbm.at[l, pl.ds(c * CK, CK)], m_v)

        def inner(gb, carry2):
            base = gb * (LANES * GB)
            xm, mv, mapped = [], [], []
            for g in range(GB):
                off = base + g * LANES
                m = m_v[pl.ds(off, LANES)]
                feats = [x_v[f, pl.ds(off, LANES)] for f in range(6)]
                xm.append([feats[f] * m for f in range(5)])
                z = (feats[5] * m).astype(jnp.int32)
                mapped.append(jnp.where((z >= 57) & (z <= 80), z - 56, 0))
                mv.append(m)
            for o in range(O_LIN):
                bo = br_v[pl.ds(o * LANES, LANES)]
                w = [wb_v[pl.ds((o * 5 + f) * LANES, LANES)] for f in range(5)]
                for g in range(GB):
                    acc = bo
                    for f in range(5):
                        acc = acc + xm[g][f] * w[f]
                    o_v[o, pl.ds(base + g * LANES, LANES)] = jnp.maximum(acc, 0.0)
            for g in range(GB):
                m8 = mapped[g] * O_EMB
                for j in range(O_EMB):
                    e = plsc.load_gather(tab_v, [m8 + j])
                    o_v[O_LIN + j, pl.ds(base + g * LANES, LANES)] = e * mv[g]
            return carry2

        lax.fori_loop(0, NGB, inner, 0)
        pltpu.sync_copy(o_v, out_hbm.at[l, :, pl.ds(c * CK, CK)])
        return carry

    lax.fori_loop(0, PER_TILE, subtask, 0)


def kernel(elements_info, elements_mask, W, b, tm_table):
    x_t = jnp.transpose(elements_info, (1, 2, 0))
    m_t = jnp.transpose(elements_mask, (1, 0))
    wb = jnp.repeat(W.reshape(-1), LANES)
    br = jnp.repeat(b, LANES)
    tab = jnp.pad(tm_table.reshape(-1), (0, 56))
    mesh = plsc.VectorSubcoreMesh(core_axis_name="c", subcore_axis_name="s")
    out = pl.kernel(
        _sc_body,
        out_type=jax.ShapeDtypeStruct((L, O, B), jnp.float32),
        mesh=mesh,
        compiler_params=pltpu.CompilerParams(needs_layout_passes=False),
        scratch_types=[
            pltpu.VMEM((F, CK), jnp.float32),
            pltpu.VMEM((CK,), jnp.float32),
            pltpu.VMEM((80 * LANES,), jnp.float32),
            pltpu.VMEM((O_LIN * LANES,), jnp.float32),
            pltpu.VMEM((256,), jnp.float32),
            pltpu.VMEM((O, CK), jnp.float32),
        ],
    )(x_t, m_t, wb, br, tab)
    return jnp.transpose(out, (2, 0, 1))
